# baseline (speedup 1.0000x reference)
import functools

import jax
import jax.numpy as jnp
from jax import lax
from jax.experimental import pallas as pl
from jax.experimental.pallas import tpu as pltpu
from jax.experimental.pallas import tpu_sc as plsc

_PATCH = 32
_HALF = 16
_NSIDE = 7
_N = _NSIDE ** 3
_K = 8
_TAU = 2.0 / 3.0
_NEG_INF = float("-inf")

_LANES = 16
_ROWS_PER_PATCH = (_PATCH * _PATCH * _PATCH) // _LANES


def _topk_kernel(k_ref, logp_ref, soft_ref, st_ref, idx_ref):
    B, N = logp_ref.shape
    kk = k_ref[0]
    noisy = logp_ref[:, :]
    lane = lax.broadcasted_iota(jnp.int32, (B, N), 1)
    r = lax.broadcasted_iota(jnp.int32, (B, _ROWS_PER_PATCH), 1)
    pattern = (r // 64) * 1024 + ((r % 64) // 2) * 8 + (r % 2)
    softs = []
    for i in range(_K):
        if i > 0:
            prev = softs[i - 1]
            m = jnp.max(prev, axis=1, keepdims=True)
            idx = jnp.min(jnp.where(prev == m, lane, N), axis=1, keepdims=True)
            masked = jnp.where(lane == idx, _NEG_INF, noisy)
            noisy = jnp.where(i < kk, masked, noisy)
        soft_i = jax.nn.softmax(noisy / _TAU, axis=1)
        soft_i = jnp.where(i < kk, soft_i, jnp.zeros_like(soft_i))
        softs.append(soft_i)
        soft_ref[2 * i:2 * i + 2, :] = soft_i
        m2 = jnp.max(soft_i, axis=1, keepdims=True)
        idx2 = jnp.min(jnp.where(soft_i == m2, lane, N), axis=1, keepdims=True)
        hard = jnp.where(lane == idx2, jnp.float32(1.0), jnp.float32(0.0))
        st_ref[2 * i:2 * i + 2, :] = (hard - soft_i) + soft_i
        d16 = idx2 // (_NSIDE * _NSIDE)
        h7 = (idx2 // _NSIDE) % _NSIDE
        w7 = idx2 % _NSIDE
        b_col = lax.broadcasted_iota(jnp.int32, (B, 1), 0)
        base = b_col * 131072 + d16 * 16384 + h7 * 128 + w7
        idx_ref[2 * i:2 * i + 2, :] = base + pattern


def _sc_gather(table_hbm, idx_hbm, out_hbm, idx_v, rows_v, sem):
    info = plsc.get_sparse_core_info()
    wid = lax.axis_index("s") * info.num_cores + lax.axis_index("c")
    nw = info.num_cores * info.num_subcores
    rows_per_w = (_K * 2 * _ROWS_PER_PATCH) // nw
    base = wid * rows_per_w
    pltpu.sync_copy(idx_hbm.at[pl.ds(base, rows_per_w)], idx_v)
    pltpu.async_copy(table_hbm.at[idx_v], rows_v, sem).wait()
    pltpu.sync_copy(rows_v, out_hbm.at[pl.ds(base, rows_per_w)])


def kernel(volume, objectness_logits, k):
    B, C, D, H, W = volume.shape
    log_p = objectness_logits.reshape(B, -1).astype(jnp.float32)
    k_arr = jnp.reshape(k, (1,)).astype(jnp.int32)

    soft16, st16, idx16 = pl.pallas_call(
        _topk_kernel,
        in_specs=[
            pl.BlockSpec(memory_space=pltpu.SMEM),
            pl.BlockSpec(memory_space=pltpu.VMEM),
        ],
        out_specs=[
            pl.BlockSpec(memory_space=pltpu.VMEM),
            pl.BlockSpec(memory_space=pltpu.VMEM),
            pl.BlockSpec(memory_space=pltpu.VMEM),
        ],
        out_shape=[
            jax.ShapeDtypeStruct((_K * B, _N), jnp.float32),
            jax.ShapeDtypeStruct((_K * B, _N), jnp.float32),
            jax.ShapeDtypeStruct((_K * B, _ROWS_PER_PATCH), jnp.int32),
        ],
    )(k_arr, log_p)

    npatch = _K * B
    n_rows = npatch * _ROWS_PER_PATCH
    table = volume.reshape(B * C * D * H * W // _LANES, _LANES)
    idx_flat = idx16.reshape(n_rows)

    info = plsc.get_sparse_core_info()
    nw = info.num_cores * info.num_subcores
    rows_per_w = n_rows // nw

    selected_rows = pl.kernel(
        _sc_gather,
        out_type=jax.ShapeDtypeStruct((n_rows, _LANES), jnp.float32),
        mesh=plsc.VectorSubcoreMesh(core_axis_name="c", subcore_axis_name="s"),
        scratch_types=[
            pltpu.VMEM((rows_per_w,), jnp.int32),
            pltpu.VMEM((rows_per_w, _LANES), jnp.float32),
            pltpu.SemaphoreType.DMA,
        ],
        compiler_params=pltpu.CompilerParams(use_tc_tiling_on_sc=False,
                                             disable_bounds_checks=True,
                                             disable_semaphore_checks=True),
    )(table, idx_flat)

    selected = selected_rows.reshape(npatch, C, _PATCH, _PATCH, _PATCH)
    soft = soft16.reshape(_K, B, _N)
    st = st16.reshape(_K, B, _N)
    return (selected, st, soft)
